# Initial kernel scaffold; baseline (speedup 1.0000x reference)
import jax
import jax.numpy as jnp
from jax.experimental import pallas as pl


def kernel(idx, nlp_tables, attention, attention2, fc_W, fc_b, fc2_W, fc2_b, cat_embedding):
    raise NotImplementedError("write your pallas kernel here")


# speedup vs baseline: 332.8114x; 332.8114x over previous
import functools

import jax
import jax.numpy as jnp
from jax import lax
from jax.experimental import pallas as pl
from jax.experimental.pallas import tpu as pltpu
from jax.experimental.pallas import tpu_sc as plsc

_NTAB = 9
_HID = 768
_VOCAB = 100000
_B = 4096
_NHEAD = 5
_EMB = 64

_NC, _NS = 2, 16
_NW = _NC * _NS
_BPW = _B // _NW
_GB = 8
_NCHUNK = _BPW // _GB
_ROWS = _NTAB * _GB
_NV = _HID // 16


def _sc_pool(tables_flat, idx9, w48):
    mesh = plsc.VectorSubcoreMesh(core_axis_name="c", subcore_axis_name="s")

    @functools.partial(
        pl.kernel,
        mesh=mesh,
        out_type=jax.ShapeDtypeStruct((_B, _NHEAD * _HID), jnp.float32),
        scratch_types=[
            pltpu.VMEM((_NCHUNK, _ROWS), jnp.int32),
            pltpu.VMEM((_ROWS, _HID), jnp.float32),
            pltpu.VMEM((_GB, _NHEAD * _HID), jnp.float32),
            pltpu.VMEM((48, 16), jnp.float32),
            pltpu.SemaphoreType.DMA,
        ],
    )
    def k(tab_hbm, idx_hbm, w_hbm, out_hbm, idx_v, rows_v, out_v, w_v, sem):
        wid = lax.axis_index("s") * _NC + lax.axis_index("c")
        base = wid * _BPW
        pltpu.sync_copy(idx_hbm.at[wid], idx_v)
        pltpu.sync_copy(w_hbm, w_v)
        wvec = [
            [w_v[h * _NTAB + i, :] for i in range(_NTAB)]
            for h in range(_NHEAD)
        ]

        def chunk_body(c, carry):
            pltpu.async_copy(tab_hbm.at[idx_v.at[c]], rows_v, sem).wait()

            def e_body(e, c2):
                def v_body(v, c3):
                    sl = pl.ds(v * 16, 16)
                    g = [rows_v[i * _GB + e, sl] for i in range(_NTAB)]
                    for h in range(_NHEAD):
                        acc = g[0] * wvec[h][0]
                        for i in range(1, _NTAB):
                            acc = acc + g[i] * wvec[h][i]
                        out_v[e, pl.ds(h * _HID + v * 16, 16)] = acc
                    return c3

                return lax.fori_loop(0, _NV, v_body, c2)

            lax.fori_loop(0, _GB, e_body, 0)
            pltpu.sync_copy(out_v, out_hbm.at[pl.ds(base + c * _GB, _GB)])
            return carry

        lax.fori_loop(0, _NCHUNK, chunk_body, 0)

    return k(tables_flat, idx9, w48)


def _mm_body(x_ref, w_ref, b_ref, o_ref):
    x = x_ref[...]
    w = w_ref[...]
    b = b_ref[...]
    for h in range(_NHEAD):
        o_ref[:, h * _EMB:(h + 1) * _EMB] = (
            jnp.dot(x[:, h * _HID:(h + 1) * _HID],
                    w[h * _HID:(h + 1) * _HID, :],
                    preferred_element_type=jnp.float32)
            + b[:, h * _EMB:(h + 1) * _EMB])


def _head_body(x_ref, ce_ref, o_ref):
    x = x_ref[...]
    ce = ce_ref[...]
    norm = jnp.sqrt(jnp.sum(x * x))
    cn = jnp.sqrt(jnp.sum(ce * ce, axis=1))
    cos = lax.dot_general(x, ce, (((1,), (1,)), ((), ())),
                          preferred_element_type=jnp.float32)
    cos = cos / norm / cn[None, :]
    m = jnp.max(cos, axis=1, keepdims=True)
    e = jnp.exp(cos - m)
    o_ref[...] = e / jnp.sum(e, axis=1, keepdims=True)


def kernel(idx, nlp_tables, attention, attention2, fc_W, fc_b, fc2_W, fc2_b,
           cat_embedding):
    tables_flat = nlp_tables.reshape(_NTAB * _VOCAB, _HID)
    offs = jnp.arange(_NTAB, dtype=jnp.int32) * _VOCAB
    idx9 = offs[:, None] + idx.astype(jnp.int32)[None, :]
    idx9 = (idx9.reshape(_NTAB, _NW, _NCHUNK, _GB)
            .transpose(1, 2, 0, 3)
            .reshape(_NW, _NCHUNK, _ROWS))
    a5 = jnp.concatenate([attention, attention2[None, :]], axis=0)
    w48 = jnp.broadcast_to(
        jnp.pad(a5.reshape(-1), (0, 48 - _NHEAD * _NTAB))[:, None], (48, 16))

    pooled = _sc_pool(tables_flat, idx9, w48)

    w_all = jnp.concatenate([fc_W, fc2_W[None]], axis=0).reshape(
        _NHEAD * _HID, _EMB)
    b_all = jnp.concatenate([fc_b, fc2_b[None]], axis=0).reshape(
        1, _NHEAD * _EMB)

    feats = pl.pallas_call(
        _mm_body,
        grid=(8,),
        in_specs=[
            pl.BlockSpec((_B // 8, _NHEAD * _HID), lambda i: (i, 0)),
            pl.BlockSpec((_NHEAD * _HID, _EMB), lambda i: (0, 0)),
            pl.BlockSpec((1, _NHEAD * _EMB), lambda i: (0, 0)),
        ],
        out_specs=pl.BlockSpec((_B // 8, _NHEAD * _EMB), lambda i: (i, 0)),
        out_shape=jax.ShapeDtypeStruct((_B, _NHEAD * _EMB), jnp.float32),
    )(pooled, w_all, b_all)

    category = pl.pallas_call(
        _head_body,
        in_specs=[
            pl.BlockSpec((_B, _EMB), lambda: (0, 0)),
            pl.BlockSpec((4, _EMB), lambda: (0, 0)),
        ],
        out_specs=pl.BlockSpec((_B, 4), lambda: (0, 0)),
        out_shape=jax.ShapeDtypeStruct((_B, 4), jnp.float32),
    )(feats[:, 4 * _EMB:], cat_embedding)

    return (feats[:, 0:_EMB], feats[:, _EMB:2 * _EMB],
            feats[:, 2 * _EMB:3 * _EMB], feats[:, 3 * _EMB:4 * _EMB],
            category)


# speedup vs baseline: 403.5279x; 1.2125x over previous
import functools

import jax
import jax.numpy as jnp
from jax import lax
from jax.experimental import pallas as pl
from jax.experimental.pallas import tpu as pltpu
from jax.experimental.pallas import tpu_sc as plsc

_NTAB = 9
_HID = 768
_VOCAB = 100000
_B = 4096
_NHEAD = 5
_EMB = 64

_NC, _NS = 2, 16
_NW = _NC * _NS
_BPW = _B // _NW
_GB = 4
_NCHUNK = _BPW // _GB
_ROWS = _NTAB * _GB
_NV = _HID // 16


def _sc_pool(tables_flat, idx9, w48):
    mesh = plsc.VectorSubcoreMesh(core_axis_name="c", subcore_axis_name="s")

    @functools.partial(
        pl.kernel,
        mesh=mesh,
        out_type=jax.ShapeDtypeStruct((_B, _NHEAD * _HID), jnp.float32),
        scratch_types=[
            pltpu.VMEM((_NCHUNK, _ROWS), jnp.int32),
            pltpu.VMEM((_ROWS, _HID), jnp.float32),
            pltpu.VMEM((_ROWS, _HID), jnp.float32),
            pltpu.VMEM((_GB, _NHEAD * _HID), jnp.float32),
            pltpu.VMEM((_GB, _NHEAD * _HID), jnp.float32),
            pltpu.VMEM((48, 16), jnp.float32),
            pltpu.SemaphoreType.DMA,
            pltpu.SemaphoreType.DMA,
            pltpu.SemaphoreType.DMA,
            pltpu.SemaphoreType.DMA,
        ],
    )
    def k(tab_hbm, idx_hbm, w_hbm, out_hbm, idx_v, rows0, rows1, out0, out1,
          w_s, g0, g1, o0, o1):
        wid = lax.axis_index("s") * _NC + lax.axis_index("c")
        base = wid * _BPW
        pltpu.sync_copy(idx_hbm.at[wid], idx_v)
        pltpu.sync_copy(w_hbm, w_s)
        w = [[w_s[h * _NTAB + i, :] for i in range(_NTAB)]
             for h in range(_NHEAD)]
        rows = (rows0, rows1)
        outs = (out0, out1)
        gsem = (g0, g1)
        osem = (o0, o1)

        pltpu.async_copy(tab_hbm.at[idx_v.at[0]], rows0, g0)
        pltpu.async_copy(tab_hbm.at[idx_v.at[1]], rows1, g1)

        def pool_chunk(rb, ob):
            def v_body(v, c3):
                sl = pl.ds(v * 16, 16)
                for e in range(_GB):
                    g = [rb[i * _GB + e, sl] for i in range(_NTAB)]
                    for h in range(_NHEAD):
                        acc = g[0] * w[h][0]
                        for i in range(1, _NTAB):
                            acc = acc + g[i] * w[h][i]
                        ob[e, pl.ds(h * _HID + v * 16, 16)] = acc
                return c3

            lax.fori_loop(0, _NV, v_body, 0)

        def pair_body(cc, carry):
            for b in range(2):
                c = 2 * cc + b
                pltpu.make_async_copy(
                    tab_hbm.at[idx_v.at[c]], rows[b], gsem[b]).wait()

                @pl.when(cc > 0)
                def _wait_out(b=b):
                    pltpu.make_async_copy(
                        outs[b], out_hbm.at[pl.ds(base, _GB)], osem[b]).wait()

                pool_chunk(rows[b], outs[b])
                pltpu.async_copy(
                    outs[b], out_hbm.at[pl.ds(base + c * _GB, _GB)], osem[b])

                @pl.when(c + 2 < _NCHUNK)
                def _next_gather(b=b, c=c):
                    pltpu.async_copy(
                        tab_hbm.at[idx_v.at[c + 2]], rows[b], gsem[b])
            return carry

        lax.fori_loop(0, _NCHUNK // 2, pair_body, 0)
        pltpu.make_async_copy(out0, out_hbm.at[pl.ds(base, _GB)], o0).wait()
        pltpu.make_async_copy(out1, out_hbm.at[pl.ds(base, _GB)], o1).wait()

    return k(tables_flat, idx9, w48)


def _mm_body(x_ref, w_ref, b_ref, o_ref):
    x = x_ref[...]
    w = w_ref[...]
    b = b_ref[...]
    for h in range(_NHEAD):
        o_ref[:, h * _EMB:(h + 1) * _EMB] = (
            jnp.dot(x[:, h * _HID:(h + 1) * _HID],
                    w[h * _HID:(h + 1) * _HID, :],
                    preferred_element_type=jnp.float32)
            + b[:, h * _EMB:(h + 1) * _EMB])


def _head_body(x_ref, ce_ref, o_ref):
    x = x_ref[...]
    ce = ce_ref[...]
    norm = jnp.sqrt(jnp.sum(x * x))
    cn = jnp.sqrt(jnp.sum(ce * ce, axis=1))
    cos = lax.dot_general(x, ce, (((1,), (1,)), ((), ())),
                          preferred_element_type=jnp.float32)
    cos = cos / norm / cn[None, :]
    m = jnp.max(cos, axis=1, keepdims=True)
    e = jnp.exp(cos - m)
    o_ref[...] = e / jnp.sum(e, axis=1, keepdims=True)


def kernel(idx, nlp_tables, attention, attention2, fc_W, fc_b, fc2_W, fc2_b,
           cat_embedding):
    tables_flat = nlp_tables.reshape(_NTAB * _VOCAB, _HID)
    offs = jnp.arange(_NTAB, dtype=jnp.int32) * _VOCAB
    idx9 = offs[:, None] + idx.astype(jnp.int32)[None, :]
    idx9 = (idx9.reshape(_NTAB, _NW, _NCHUNK, _GB)
            .transpose(1, 2, 0, 3)
            .reshape(_NW, _NCHUNK, _ROWS))
    a5 = jnp.concatenate([attention, attention2[None, :]], axis=0)
    w48 = jnp.broadcast_to(
        jnp.pad(a5.reshape(-1), (0, 48 - _NHEAD * _NTAB))[:, None], (48, 16))

    pooled = _sc_pool(tables_flat, idx9, w48)

    w_all = jnp.concatenate([fc_W, fc2_W[None]], axis=0).reshape(
        _NHEAD * _HID, _EMB)
    b_all = jnp.concatenate([fc_b, fc2_b[None]], axis=0).reshape(
        1, _NHEAD * _EMB)

    feats = pl.pallas_call(
        _mm_body,
        grid=(8,),
        in_specs=[
            pl.BlockSpec((_B // 8, _NHEAD * _HID), lambda i: (i, 0)),
            pl.BlockSpec((_NHEAD * _HID, _EMB), lambda i: (0, 0)),
            pl.BlockSpec((1, _NHEAD * _EMB), lambda i: (0, 0)),
        ],
        out_specs=pl.BlockSpec((_B // 8, _NHEAD * _EMB), lambda i: (i, 0)),
        out_shape=jax.ShapeDtypeStruct((_B, _NHEAD * _EMB), jnp.float32),
    )(pooled, w_all, b_all)

    category = pl.pallas_call(
        _head_body,
        in_specs=[
            pl.BlockSpec((_B, _EMB), lambda: (0, 0)),
            pl.BlockSpec((4, _EMB), lambda: (0, 0)),
        ],
        out_specs=pl.BlockSpec((_B, 4), lambda: (0, 0)),
        out_shape=jax.ShapeDtypeStruct((_B, 4), jnp.float32),
    )(feats[:, 4 * _EMB:], cat_embedding)

    return (feats[:, 0:_EMB], feats[:, _EMB:2 * _EMB],
            feats[:, 2 * _EMB:3 * _EMB], feats[:, 3 * _EMB:4 * _EMB],
            category)
